# Initial kernel scaffold; baseline (speedup 1.0000x reference)
import jax
import jax.numpy as jnp
from jax.experimental import pallas as pl


def kernel(hidden_states, w_pre_ln, wqkv, wo, w_post_ln, w_gate, w1, w2):
    raise NotImplementedError("write your pallas kernel here")


# speedup vs baseline: 1.1242x; 1.1242x over previous
import functools

import jax
import jax.numpy as jnp
from jax.experimental import pallas as pl
from jax.experimental.pallas import tpu as pltpu

_T, _D, _H, _HD, _E, _K, _F = 2048, 1024, 16, 64, 8, 2, 768
_EPS = 1e-6


def _qkv_kernel(hid_ref, wln_ref, wqkv_ref, qkv_ref):
    x = hid_ref[:]
    var = jnp.mean(x * x, axis=-1, keepdims=True)
    h = x * jax.lax.rsqrt(var + _EPS) * wln_ref[:]
    qkv_ref[:] = jnp.dot(h.astype(jnp.bfloat16), wqkv_ref[:],
                         preferred_element_type=jnp.float32)


def _attn_kernel(q_ref, k_ref, v_ref, o_ref, *, bq, bk):
    i = pl.program_id(1)
    q = q_ref[0]
    scale = jnp.float32(1.0 / (_HD ** 0.5))
    qpos = i * bq + jax.lax.broadcasted_iota(jnp.int32, (bq, bk), 0)

    def body(j, carry):
        m, l, acc = carry
        k = k_ref[0, pl.ds(j * bk, bk), :]
        s = jax.lax.dot_general(q, k, (((1,), (1,)), ((), ())),
                                preferred_element_type=jnp.float32) * scale
        kpos = j * bk + jax.lax.broadcasted_iota(jnp.int32, (bq, bk), 1)
        s = jnp.where(qpos >= kpos, s, -1e30)
        m_new = jnp.maximum(m, jnp.max(s, axis=-1, keepdims=True))
        p = jnp.exp(s - m_new)
        alpha = jnp.exp(m - m_new)
        l = l * alpha + jnp.sum(p, axis=-1, keepdims=True)
        vj = v_ref[0, pl.ds(j * bk, bk), :]
        acc = acc * alpha + jnp.dot(p.astype(jnp.bfloat16), vj,
                                    preferred_element_type=jnp.float32)
        return m_new, l, acc

    m0 = jnp.full((bq, 1), -1e30, jnp.float32)
    l0 = jnp.zeros((bq, 1), jnp.float32)
    a0 = jnp.zeros((bq, _HD), jnp.float32)
    m, l, acc = jax.lax.fori_loop(0, i + 1, body, (m0, l0, a0))
    o_ref[0] = (acc / l).astype(o_ref.dtype)


def _wo_router_kernel(attn_ref, wo_ref, res_ref, wln_ref, wgate_ref,
                      hid2_ref, h2_ref, comb_ref):
    y = jnp.dot(attn_ref[:], wo_ref[:], preferred_element_type=jnp.float32)
    hid2 = res_ref[:] + y
    hid2_ref[:] = hid2
    var = jnp.mean(hid2 * hid2, axis=-1, keepdims=True)
    h2 = hid2 * jax.lax.rsqrt(var + _EPS) * wln_ref[:]
    h2_ref[:] = h2.astype(jnp.bfloat16)
    logits = jnp.dot(h2, wgate_ref[:], preferred_element_type=jnp.float32)
    lmax = jnp.max(logits, axis=-1, keepdims=True)
    ex = jnp.exp(logits - lmax)
    p = ex / jnp.sum(ex, axis=-1, keepdims=True)
    colid = jax.lax.broadcasted_iota(jnp.int32, p.shape, 1)
    m1 = jnp.max(p, axis=-1, keepdims=True)
    i1 = jnp.min(jnp.where(p == m1, colid, _E), axis=-1, keepdims=True)
    sel1 = colid == i1
    pm = jnp.where(sel1, -1.0, p)
    m2 = jnp.max(pm, axis=-1, keepdims=True)
    i2 = jnp.min(jnp.where(pm == m2, colid, _E), axis=-1, keepdims=True)
    sel2 = colid == i2
    ws = m1 + m2
    comb_ref[:] = (jnp.where(sel1, m1 / ws, 0.0)
                   + jnp.where(sel2, m2 / ws, 0.0))


def _moe_kernel(h2_ref, w1_ref, w2_ref, comb_ref, hid2_ref, out_ref):
    e = pl.program_id(1)

    @pl.when(e == 0)
    def _():
        out_ref[:] = hid2_ref[:]

    gu = jnp.dot(h2_ref[:], w1_ref[0], preferred_element_type=jnp.float32)
    g = gu[:, :_F]
    u = gu[:, _F:]
    act = (g * jax.nn.sigmoid(g) * u).astype(jnp.bfloat16)
    y = jnp.dot(act, w2_ref[0], preferred_element_type=jnp.float32)
    colid = jax.lax.broadcasted_iota(jnp.int32, comb_ref.shape, 1)
    we = jnp.sum(jnp.where(colid == e, comb_ref[:], 0.0), axis=-1,
                 keepdims=True)
    out_ref[:] = out_ref[:] + we * y


def kernel(hidden_states, w_pre_ln, wqkv, wo, w_post_ln, w_gate, w1, w2):
    f32, bf16 = jnp.float32, jnp.bfloat16
    bt = 512
    qkv = pl.pallas_call(
        _qkv_kernel,
        grid=(_T // bt,),
        in_specs=[
            pl.BlockSpec((bt, _D), lambda i: (i, 0)),
            pl.BlockSpec((_D,), lambda i: (0,)),
            pl.BlockSpec((_D, 3 * _D), lambda i: (0, 0)),
        ],
        out_specs=pl.BlockSpec((bt, 3 * _D), lambda i: (i, 0)),
        out_shape=jax.ShapeDtypeStruct((_T, 3 * _D), f32),
    )(hidden_states, w_pre_ln, wqkv.astype(bf16))

    q = qkv[:, :_D].reshape(_T, _H, _HD).transpose(1, 0, 2).astype(bf16)
    k = qkv[:, _D:2 * _D].reshape(_T, _H, _HD).transpose(1, 0, 2).astype(bf16)
    v = qkv[:, 2 * _D:].reshape(_T, _H, _HD).transpose(1, 0, 2).astype(bf16)

    bq, bk = 256, 256
    attn = pl.pallas_call(
        functools.partial(_attn_kernel, bq=bq, bk=bk),
        grid=(_H, _T // bq),
        in_specs=[
            pl.BlockSpec((1, bq, _HD), lambda h, i: (h, i, 0)),
            pl.BlockSpec((1, _T, _HD), lambda h, i: (h, 0, 0)),
            pl.BlockSpec((1, _T, _HD), lambda h, i: (h, 0, 0)),
        ],
        out_specs=pl.BlockSpec((1, bq, _HD), lambda h, i: (h, i, 0)),
        out_shape=jax.ShapeDtypeStruct((_H, _T, _HD), bf16),
        compiler_params=pltpu.CompilerParams(
            dimension_semantics=("parallel", "arbitrary")),
    )(q, k, v)
    attn = attn.transpose(1, 0, 2).reshape(_T, _D)

    bt2 = 256
    hid2, h2, comb = pl.pallas_call(
        _wo_router_kernel,
        grid=(_T // bt2,),
        in_specs=[
            pl.BlockSpec((bt2, _D), lambda i: (i, 0)),
            pl.BlockSpec((_D, _D), lambda i: (0, 0)),
            pl.BlockSpec((bt2, _D), lambda i: (i, 0)),
            pl.BlockSpec((_D,), lambda i: (0,)),
            pl.BlockSpec((_D, _E), lambda i: (0, 0)),
        ],
        out_specs=[
            pl.BlockSpec((bt2, _D), lambda i: (i, 0)),
            pl.BlockSpec((bt2, _D), lambda i: (i, 0)),
            pl.BlockSpec((bt2, _E), lambda i: (i, 0)),
        ],
        out_shape=[
            jax.ShapeDtypeStruct((_T, _D), f32),
            jax.ShapeDtypeStruct((_T, _D), bf16),
            jax.ShapeDtypeStruct((_T, _E), f32),
        ],
    )(attn, wo.astype(bf16), hidden_states, w_post_ln, w_gate)

    btm = 1024
    out = pl.pallas_call(
        _moe_kernel,
        grid=(_T // btm, _E),
        in_specs=[
            pl.BlockSpec((btm, _D), lambda i, e: (i, 0)),
            pl.BlockSpec((1, _D, 2 * _F), lambda i, e: (e, 0, 0)),
            pl.BlockSpec((1, _F, _D), lambda i, e: (e, 0, 0)),
            pl.BlockSpec((btm, _E), lambda i, e: (i, 0)),
            pl.BlockSpec((btm, _D), lambda i, e: (i, 0)),
        ],
        out_specs=pl.BlockSpec((btm, _D), lambda i, e: (i, 0)),
        out_shape=jax.ShapeDtypeStruct((_T, _D), f32),
        compiler_params=pltpu.CompilerParams(
            dimension_semantics=("parallel", "arbitrary")),
    )(h2, w1.astype(bf16), w2.astype(bf16), comb, hid2)
    return out


# speedup vs baseline: 1.6791x; 1.4935x over previous
import functools

import jax
import jax.numpy as jnp
from jax.experimental import pallas as pl
from jax.experimental.pallas import tpu as pltpu

_T, _D, _H, _HD, _E, _K, _F = 2048, 1024, 16, 64, 8, 2, 768
_EPS = 1e-6


def _qkv_kernel(hid_ref, wln_ref, wqkv_ref, qkv_ref):
    x = hid_ref[:]
    var = jnp.mean(x * x, axis=-1, keepdims=True)
    h = x * jax.lax.rsqrt(var + _EPS) * wln_ref[:]
    qkv_ref[:] = jnp.dot(h.astype(jnp.bfloat16), wqkv_ref[:],
                         preferred_element_type=jnp.float32
                         ).astype(jnp.bfloat16)


def _attn_kernel(q_ref, k_ref, v_ref, o_ref, *, bq, bk):
    i = pl.program_id(1)
    q = q_ref[0]
    scale = jnp.float32(1.0 / (_HD ** 0.5))

    def step(j, carry, masked):
        m, l, acc = carry
        k = k_ref[0, pl.ds(j * bk, bk), :]
        s = jax.lax.dot_general(q, k, (((1,), (1,)), ((), ())),
                                preferred_element_type=jnp.float32) * scale
        if masked:
            qpos = i * bq + jax.lax.broadcasted_iota(jnp.int32, (bq, bk), 0)
            kpos = j * bk + jax.lax.broadcasted_iota(jnp.int32, (bq, bk), 1)
            s = jnp.where(qpos >= kpos, s, -1e30)
        m_new = jnp.maximum(m, jnp.max(s, axis=-1, keepdims=True))
        p = jnp.exp(s - m_new)
        alpha = jnp.exp(m - m_new)
        l = l * alpha + jnp.sum(p, axis=-1, keepdims=True)
        vj = v_ref[0, pl.ds(j * bk, bk), :]
        acc = acc * alpha + jnp.dot(p.astype(jnp.bfloat16), vj,
                                    preferred_element_type=jnp.float32)
        return m_new, l, acc

    m0 = jnp.full((bq, 1), -1e30, jnp.float32)
    l0 = jnp.zeros((bq, 1), jnp.float32)
    a0 = jnp.zeros((bq, _HD), jnp.float32)
    carry = jax.lax.fori_loop(0, i, lambda j, c: step(j, c, False),
                              (m0, l0, a0))
    _, l, acc = step(i, carry, True)
    o_ref[0] = (acc / l).astype(o_ref.dtype)


def _wo_router_kernel(attn_ref, wo_ref, res_ref, wln_ref, wgate_ref,
                      hid2_ref, h2_ref, comb_ref):
    y = jnp.dot(attn_ref[:], wo_ref[:], preferred_element_type=jnp.float32)
    hid2 = res_ref[:] + y
    hid2_ref[:] = hid2
    var = jnp.mean(hid2 * hid2, axis=-1, keepdims=True)
    h2 = hid2 * jax.lax.rsqrt(var + _EPS) * wln_ref[:]
    h2_ref[:] = h2.astype(jnp.bfloat16)
    logits = jnp.dot(h2, wgate_ref[:], preferred_element_type=jnp.float32)
    lmax = jnp.max(logits, axis=-1, keepdims=True)
    ex = jnp.exp(logits - lmax)
    p = ex / jnp.sum(ex, axis=-1, keepdims=True)
    colid = jax.lax.broadcasted_iota(jnp.int32, p.shape, 1)
    m1 = jnp.max(p, axis=-1, keepdims=True)
    i1 = jnp.min(jnp.where(p == m1, colid, _E), axis=-1, keepdims=True)
    sel1 = colid == i1
    pm = jnp.where(sel1, -1.0, p)
    m2 = jnp.max(pm, axis=-1, keepdims=True)
    i2 = jnp.min(jnp.where(pm == m2, colid, _E), axis=-1, keepdims=True)
    sel2 = colid == i2
    ws = m1 + m2
    comb_ref[:] = (jnp.where(sel1, m1 / ws, 0.0)
                   + jnp.where(sel2, m2 / ws, 0.0))


def _moe_kernel(h2_ref, w1_ref, w2_ref, comb_ref, hid2_ref, out_ref):
    e = pl.program_id(1)

    @pl.when(e == 0)
    def _():
        out_ref[:] = hid2_ref[:]

    gu = jnp.dot(h2_ref[:], w1_ref[0], preferred_element_type=jnp.float32)
    g = gu[:, :_F]
    u = gu[:, _F:]
    act = (g * jax.nn.sigmoid(g) * u).astype(jnp.bfloat16)
    y = jnp.dot(act, w2_ref[0], preferred_element_type=jnp.float32)
    colid = jax.lax.broadcasted_iota(jnp.int32, comb_ref.shape, 1)
    we = jnp.sum(jnp.where(colid == e, comb_ref[:], 0.0), axis=-1,
                 keepdims=True)
    out_ref[:] = out_ref[:] + we * y


def kernel(hidden_states, w_pre_ln, wqkv, wo, w_post_ln, w_gate, w1, w2):
    f32, bf16 = jnp.float32, jnp.bfloat16
    bt = 512
    qkv = pl.pallas_call(
        _qkv_kernel,
        grid=(_T // bt,),
        in_specs=[
            pl.BlockSpec((bt, _D), lambda i: (i, 0)),
            pl.BlockSpec((_D,), lambda i: (0,)),
            pl.BlockSpec((_D, 3 * _D), lambda i: (0, 0)),
        ],
        out_specs=pl.BlockSpec((bt, 3 * _D), lambda i: (i, 0)),
        out_shape=jax.ShapeDtypeStruct((_T, 3 * _D), bf16),
    )(hidden_states, w_pre_ln, wqkv.astype(bf16))

    q = qkv[:, :_D].reshape(_T, _H, _HD).transpose(1, 0, 2)
    k = qkv[:, _D:2 * _D].reshape(_T, _H, _HD).transpose(1, 0, 2)
    v = qkv[:, 2 * _D:].reshape(_T, _H, _HD).transpose(1, 0, 2)

    bq, bk = 512, 512
    attn = pl.pallas_call(
        functools.partial(_attn_kernel, bq=bq, bk=bk),
        grid=(_H, _T // bq),
        in_specs=[
            pl.BlockSpec((1, bq, _HD), lambda h, i: (h, i, 0)),
            pl.BlockSpec((1, _T, _HD), lambda h, i: (h, 0, 0)),
            pl.BlockSpec((1, _T, _HD), lambda h, i: (h, 0, 0)),
        ],
        out_specs=pl.BlockSpec((1, bq, _HD), lambda h, i: (h, i, 0)),
        out_shape=jax.ShapeDtypeStruct((_H, _T, _HD), bf16),
        compiler_params=pltpu.CompilerParams(
            dimension_semantics=("arbitrary", "arbitrary")),
    )(q, k, v)
    attn = attn.transpose(1, 0, 2).reshape(_T, _D)

    bt2 = 512
    hid2, h2, comb = pl.pallas_call(
        _wo_router_kernel,
        grid=(_T // bt2,),
        in_specs=[
            pl.BlockSpec((bt2, _D), lambda i: (i, 0)),
            pl.BlockSpec((_D, _D), lambda i: (0, 0)),
            pl.BlockSpec((bt2, _D), lambda i: (i, 0)),
            pl.BlockSpec((_D,), lambda i: (0,)),
            pl.BlockSpec((_D, _E), lambda i: (0, 0)),
        ],
        out_specs=[
            pl.BlockSpec((bt2, _D), lambda i: (i, 0)),
            pl.BlockSpec((bt2, _D), lambda i: (i, 0)),
            pl.BlockSpec((bt2, _E), lambda i: (i, 0)),
        ],
        out_shape=[
            jax.ShapeDtypeStruct((_T, _D), f32),
            jax.ShapeDtypeStruct((_T, _D), bf16),
            jax.ShapeDtypeStruct((_T, _E), f32),
        ],
    )(attn, wo.astype(bf16), hidden_states, w_post_ln, w_gate)

    btm = 1024
    out = pl.pallas_call(
        _moe_kernel,
        grid=(_T // btm, _E),
        in_specs=[
            pl.BlockSpec((btm, _D), lambda i, e: (i, 0)),
            pl.BlockSpec((1, _D, 2 * _F), lambda i, e: (e, 0, 0)),
            pl.BlockSpec((1, _F, _D), lambda i, e: (e, 0, 0)),
            pl.BlockSpec((btm, _E), lambda i, e: (i, 0)),
            pl.BlockSpec((btm, _D), lambda i, e: (i, 0)),
        ],
        out_specs=pl.BlockSpec((btm, _D), lambda i, e: (i, 0)),
        out_shape=jax.ShapeDtypeStruct((_T, _D), f32),
        compiler_params=pltpu.CompilerParams(
            dimension_semantics=("parallel", "arbitrary")),
    )(h2, w1.astype(bf16), w2.astype(bf16), comb, hid2)
    return out
